# Initial kernel scaffold; baseline (speedup 1.0000x reference)
import jax
import jax.numpy as jnp
from jax.experimental import pallas as pl


def kernel(idx, targets, tok_table, pos_table, W, b):
    raise NotImplementedError("write your pallas kernel here")


# speedup vs baseline: 2.4422x; 2.4422x over previous
import functools

import jax
import jax.numpy as jnp
from jax.experimental import pallas as pl

VOCAB = 1000
EMBD = 64
BATCH = 1024
TLEN = 50
NB = 16
GRID = BATCH // NB
ROWS = NB * TLEN
NTOK = BATCH * TLEN


def _fused_body(idx_ref, tgt_ref, tok_ref, pos_ref, w_ref, b_ref,
                out_ref, loss_ref):
    i = pl.program_id(0)

    vcol = jax.lax.broadcasted_iota(jnp.int32, (ROWS, VOCAB), 1)
    onehot = jnp.where(vcol == idx_ref[...], 1.0, 0.0).astype(jnp.float32)
    tok = jax.lax.dot_general(onehot, tok_ref[...],
                              (((1,), (0,)), ((), ())),
                              preferred_element_type=jnp.float32)

    x = tok + pos_ref[...]

    logits = jax.lax.dot_general(x, w_ref[...],
                                 (((1,), (0,)), ((), ())),
                                 preferred_element_type=jnp.float32)
    logits = logits + b_ref[...]
    out_ref[...] = logits

    m = jnp.max(logits, axis=1, keepdims=True)
    s = jnp.sum(jnp.exp(logits - m), axis=1, keepdims=True)
    lse = m + jnp.log(s)
    ll = jnp.sum(jnp.where(vcol == tgt_ref[...], logits, 0.0),
                 axis=1, keepdims=True)
    part = jnp.sum(lse - ll, keepdims=True)

    @pl.when(i == 0)
    def _():
        loss_ref[...] = jnp.zeros((1, 1), jnp.float32)

    loss_ref[...] += part

    @pl.when(i == GRID - 1)
    def _():
        loss_ref[...] = loss_ref[...] * (1.0 / NTOK)


@functools.partial(jax.jit, static_argnames=("interpret",))
def _fused(idx, targets, tok_table, pos_table, W, b, interpret=False):
    out_logits, out_loss = pl.pallas_call(
        _fused_body,
        grid=(GRID,),
        in_specs=[
            pl.BlockSpec((ROWS, 1), lambda i: (i, 0)),
            pl.BlockSpec((ROWS, 1), lambda i: (i, 0)),
            pl.BlockSpec((VOCAB, EMBD), lambda i: (0, 0)),
            pl.BlockSpec((ROWS, EMBD), lambda i: (0, 0)),
            pl.BlockSpec((EMBD, VOCAB), lambda i: (0, 0)),
            pl.BlockSpec((1, VOCAB), lambda i: (0, 0)),
        ],
        out_specs=[
            pl.BlockSpec((ROWS, VOCAB), lambda i: (i, 0)),
            pl.BlockSpec((1, 1), lambda i: (0, 0)),
        ],
        out_shape=[
            jax.ShapeDtypeStruct((NTOK, VOCAB), jnp.float32),
            jax.ShapeDtypeStruct((1, 1), jnp.float32),
        ],
        interpret=interpret,
    )(idx.reshape(NTOK, 1), targets.reshape(NTOK, 1), tok_table,
      jnp.tile(pos_table, (NB, 1)), W, b.reshape(1, VOCAB))
    return out_logits, out_loss[0, 0]


def kernel(idx, targets, tok_table, pos_table, W, b):
    return _fused(idx, targets, tok_table, pos_table, W, b)
